# Initial kernel scaffold; baseline (speedup 1.0000x reference)
import jax
import jax.numpy as jnp
from jax.experimental import pallas as pl


def kernel(x, edge_index, community, multi_community_nodes, multi_community_index, emb1_W, emb1_b, emb2_W, emb2_b, emb3_W, emb3_b, conv1_W, conv1_b, conv2_W, conv2_b, lin1_W, lin1_b, lin2_W, lin2_b):
    raise NotImplementedError("write your pallas kernel here")


# speedup vs baseline: 6.2096x; 6.2096x over previous
import functools

import jax
import jax.numpy as jnp
from jax import lax
from jax.experimental import pallas as pl
from jax.experimental.pallas import tpu as pltpu
from jax.experimental.pallas import tpu_sc as plsc

N = 10000
E = 320000
C = 1024
F = 128
NC = 2
NS = 16
NW = NC * NS
EPW = E // NW
KE = 80
NCH = EPW // KE
ZR = 1000
ZW = N // ZR
CPW = C // NW
W16 = 16

_mesh = plsc.VectorSubcoreMesh(core_axis_name="c", subcore_axis_name="s")


def _wid():
    return lax.axis_index("s") * NC + lax.axis_index("c")


def _sc_counts_body(dst_hbm, comm_hbm, ones_hbm, zer_hbm,
                    deg_out, cnt_out,
                    dvec, cvec, ones_v, accd, accc, sem):
    c = lax.axis_index("c")
    s = lax.axis_index("s")
    wid = _wid()
    @pl.when(s < ZW)
    def _():
        pltpu.sync_copy(zer_hbm, accd.at[pl.ds(s * ZR, ZR)])
    pltpu.sync_copy(zer_hbm.at[pl.ds(0, C // NS)],
                    accc.at[pl.ds(s * (C // NS), C // NS)])
    pltpu.sync_copy(ones_hbm, ones_v)
    plsc.subcore_barrier()

    def deg_step(k, carry):
        base = wid * EPW + k * KE
        pltpu.sync_copy(dst_hbm.at[pl.ds(base, KE)], dvec)
        pltpu.sync_copy(ones_v, accd.at[dvec], add=True)
        return carry
    lax.fori_loop(0, NCH, deg_step, 0)

    def cnt_step(k, carry):
        j = wid + k * NW

        @pl.when(j < NCH)
        def _():
            base = j * KE
            pltpu.sync_copy(comm_hbm.at[pl.ds(base, KE)], cvec)
            pltpu.sync_copy(ones_v, accc.at[cvec], add=True)
        return carry
    lax.fori_loop(0, (NCH + NW - 1) // NW, cnt_step, 0)

    plsc.subcore_barrier()

    @pl.when(s < ZW)
    def _():
        pltpu.sync_copy(accd.at[pl.ds(s * ZR, ZR)], deg_out.at[c, pl.ds(s * ZR, ZR)])
    pltpu.sync_copy(accc.at[pl.ds(s * (C // NS), C // NS)],
                    cnt_out.at[c, pl.ds(s * (C // NS), C // NS)])


_sc_counts = pl.kernel(
    _sc_counts_body,
    out_type=(jax.ShapeDtypeStruct((NC, N, F), jnp.float32),
              jax.ShapeDtypeStruct((NC, C, F), jnp.float32)),
    mesh=_mesh,
    scratch_types=[
        pltpu.VMEM((KE,), jnp.int32),
        pltpu.VMEM((KE,), jnp.int32),
        pltpu.VMEM((KE, F), jnp.float32),
        pltpu.VMEM_SHARED((N, F), jnp.float32),
        pltpu.VMEM_SHARED((C, F), jnp.float32),
        pltpu.SemaphoreType.DMA,
    ],
)


NH = N // NC
ACCR = NH + 8
EPS = E // NS
NCHA = EPS // KE


def _sc_agg_body(g_hbm, src_hbm, dst_hbm, zrows_hbm,
                 out_hbm,
                 src_v, dst_v, rows_v, acc, sem):
    c = lax.axis_index("c")
    s = lax.axis_index("s")
    half = c * NH

    @pl.when(s < ZW // NC)
    def _():
        pltpu.sync_copy(zrows_hbm, acc.at[pl.ds(s * ZR, ZR)])

    @pl.when(s == ZW // NC)
    def _():
        pltpu.sync_copy(zrows_hbm.at[pl.ds(0, 8)], acc.at[pl.ds(NH, 8)])
    plsc.subcore_barrier()

    def step(k, carry):
        base = s * EPS + k * KE
        pltpu.sync_copy(src_hbm.at[pl.ds(base, KE)], src_v)
        pltpu.sync_copy(dst_hbm.at[pl.ds(base, KE)], dst_v)
        for j in range(KE // 16):
            d16 = dst_v[pl.ds(j * 16, 16)] - half
            ok = (d16 >= 0) & (d16 < NH)
            dst_v[pl.ds(j * 16, 16)] = jnp.where(ok, d16, NH)
        pltpu.async_copy(g_hbm.at[src_v], rows_v, sem).wait()
        pltpu.sync_copy(rows_v, acc.at[dst_v], add=True)
        return carry
    lax.fori_loop(0, NCHA, step, 0)

    plsc.subcore_barrier()

    @pl.when(s < ZW // NC)
    def _():
        pltpu.sync_copy(acc.at[pl.ds(s * ZR, ZR)], out_hbm.at[c, pl.ds(s * ZR, ZR)])


_sc_agg = pl.kernel(
    _sc_agg_body,
    out_type=jax.ShapeDtypeStruct((NC, NH, F), jnp.float32),
    mesh=_mesh,
    scratch_types=[
        pltpu.VMEM((KE,), jnp.int32),
        pltpu.VMEM((KE,), jnp.int32),
        pltpu.VMEM((KE, F), jnp.float32),
        pltpu.VMEM_SHARED((ACCR, F), jnp.float32),
        pltpu.SemaphoreType.DMA,
    ],
)


FG = F // 16
Q = NW // FG
CHK = 400
NCHKT = N // CHK
KPQ = (NCHKT + Q - 1) // Q


def _sc_pool_body(ht_hbm, comm_hbm,
                  psum_out, pmax_out,
                  block_v, cvec, asum, amax, sem):
    wid = _wid()
    fg = wid % FG
    q = wid // FG

    zer = jnp.zeros((16,), jnp.float32)
    ninf = jnp.full((16,), -jnp.inf, jnp.float32)

    def init_acc(i, carry):
        asum[pl.ds(i * 16, 16)] = zer
        amax[pl.ds(i * 16, 16)] = ninf
        return carry
    lax.fori_loop(0, C, init_acc, 0)

    for k in range(KPQ):
        j = q + k * Q

        @pl.when(j < NCHKT)
        def _():
            base = j * CHK
            pltpu.sync_copy(ht_hbm.at[pl.ds(fg * N + base, CHK)], block_v)
            pltpu.sync_copy(comm_hbm.at[pl.ds(base, CHK)], cvec)

            def group(g, carry):
                cv16 = cvec[pl.ds(g * 16, 16)]
                for l in range(16):
                    a = cv16[l] * 16
                    val = block_v[g * 16 + l]
                    asum[pl.ds(a, 16)] = asum[pl.ds(a, 16)] + val
                    amax[pl.ds(a, 16)] = jnp.maximum(amax[pl.ds(a, 16)], val)
                return carry
            lax.fori_loop(0, CHK // 16, group, 0)

    pltpu.sync_copy(asum, psum_out.at[q, fg])
    pltpu.sync_copy(amax, pmax_out.at[q, fg])


_sc_pool = pl.kernel(
    _sc_pool_body,
    out_type=(jax.ShapeDtypeStruct((Q, FG, C * 16), jnp.float32),
              jax.ShapeDtypeStruct((Q, FG, C * 16), jnp.float32)),
    mesh=_mesh,
    scratch_types=[
        pltpu.VMEM((CHK, 16), jnp.float32),
        pltpu.VMEM((CHK,), jnp.int32),
        pltpu.VMEM((C * 16,), jnp.float32),
        pltpu.VMEM((C * 16,), jnp.float32),
        pltpu.SemaphoreType.DMA,
    ],
)


RB = 1000
GRID = N // RB


def _tc_prep_body(x_ref, w1, b1, w2, b2, w3, b3, wc1, deg_ref, g1_ref, dinv_ref):
    xb = x_ref[...]
    x1 = jax.nn.relu(xb[:, :8] @ w1[...] + b1[...])
    x2 = jax.nn.relu(xb[:, 8:20] @ w2[...] + b2[...])
    h = jax.nn.relu(jnp.concatenate([x1, x2], axis=1) @ w3[...] + b3[...])
    hw = h @ wc1[...]
    d3 = deg_ref[...]
    deg = d3[0, :, 0] + d3[1, :, 0] + 1.0
    dv = lax.rsqrt(deg)
    dinv_ref[...] = dv[:, None]
    g1_ref[...] = hw * dv[:, None]


def _tc_prep(x, w1, b1, w2, b2, w3, b3, wc1, deg_rows):
    return pl.pallas_call(
        _tc_prep_body,
        grid=(GRID,),
        in_specs=[
            pl.BlockSpec((RB, 20), lambda i: (i, 0)),
            pl.BlockSpec((8, F), lambda i: (0, 0)),
            pl.BlockSpec((F,), lambda i: (0,)),
            pl.BlockSpec((12, F), lambda i: (0, 0)),
            pl.BlockSpec((F,), lambda i: (0,)),
            pl.BlockSpec((2 * F, 2 * F), lambda i: (0, 0)),
            pl.BlockSpec((2 * F,), lambda i: (0,)),
            pl.BlockSpec((2 * F, F), lambda i: (0, 0)),
            pl.BlockSpec((NC, RB, F), lambda i: (0, i, 0)),
        ],
        out_specs=[
            pl.BlockSpec((RB, F), lambda i: (i, 0)),
            pl.BlockSpec((RB, 1), lambda i: (i, 0)),
        ],
        out_shape=[
            jax.ShapeDtypeStruct((N, F), jnp.float32),
            jax.ShapeDtypeStruct((N, 1), jnp.float32),
        ],
    )(x, w1, b1, w2, b2, w3, b3, wc1, deg_rows)


def _tc_mid_body(ap_ref, g1_ref, dinv_ref, bc1, wc2, h1t_ref, g2_ref):
    dv = dinv_ref[...]
    h1 = jax.nn.relu(dv * (ap_ref[...] + g1_ref[...]) + bc1[...])
    for j in range(FG):
        h1t_ref[j] = h1[:, j * 16:(j + 1) * 16]
    g2_ref[...] = (h1 @ wc2[...]) * dv


def _tc_mid(ap, g1, dinv, bc1, wc2):
    return pl.pallas_call(
        _tc_mid_body,
        grid=(GRID,),
        in_specs=[
            pl.BlockSpec((RB, F), lambda i: (i, 0)),
            pl.BlockSpec((RB, F), lambda i: (i, 0)),
            pl.BlockSpec((RB, 1), lambda i: (i, 0)),
            pl.BlockSpec((F,), lambda i: (0,)),
            pl.BlockSpec((F, F), lambda i: (0, 0)),
        ],
        out_specs=[
            pl.BlockSpec((FG, RB, 16), lambda i: (0, i, 0)),
            pl.BlockSpec((RB, F), lambda i: (i, 0)),
        ],
        out_shape=[
            jax.ShapeDtypeStruct((FG, N, 16), jnp.float32),
            jax.ShapeDtypeStruct((N, F), jnp.float32),
        ],
    )(ap, g1, dinv, bc1, wc2)


def _tc_post_body(ap_ref, g2_ref, dinv_ref, bc2, h2t_ref):
    h2 = jax.nn.relu(dinv_ref[...] * (ap_ref[...] + g2_ref[...]) + bc2[...])
    for j in range(FG):
        h2t_ref[j] = h2[:, j * 16:(j + 1) * 16]


def _tc_post(ap2, g2, dinv, bc2):
    return pl.pallas_call(
        _tc_post_body,
        grid=(GRID,),
        in_specs=[
            pl.BlockSpec((RB, F), lambda i: (i, 0)),
            pl.BlockSpec((RB, F), lambda i: (i, 0)),
            pl.BlockSpec((RB, 1), lambda i: (i, 0)),
            pl.BlockSpec((F,), lambda i: (0,)),
        ],
        out_specs=pl.BlockSpec((FG, RB, 16), lambda i: (0, i, 0)),
        out_shape=jax.ShapeDtypeStruct((FG, N, 16), jnp.float32),
    )(ap2, g2, dinv, bc2)


def _tc_merge_body(ps1, pm1, ps2, pm2, cnt_ref, mean_ref, mx_ref):
    cr = cnt_ref[...]
    cntf = cr[0] + cr[1]
    s1 = ps1[...][:, 0, 0, :].sum(axis=0)
    s2 = ps2[...][:, 0, 0, :].sum(axis=0)
    m1 = pm1[...][:, 0, 0, :].max(axis=0)
    m2 = pm2[...][:, 0, 0, :].max(axis=0)
    nz = cntf > 0.0
    mean = (s1 + s2) / jnp.maximum(cntf, 1.0)
    mx = jnp.where(nz, m1, 0.0) + jnp.where(nz, m2, 0.0)
    mean_ref[...] = mean[None, None, :]
    mx_ref[...] = mx[None, None, :]


def _tc_merge(ps1, pm1, ps2, pm2, cnt2):
    L = C * 16
    return pl.pallas_call(
        _tc_merge_body,
        grid=(FG,),
        in_specs=[pl.BlockSpec((Q, 1, 1, L), lambda j: (0, j, 0, 0))] * 4 +
                 [pl.BlockSpec((NC, L), lambda j: (0, 0))],
        out_specs=[
            pl.BlockSpec((1, 1, L), lambda j: (j, 0, 0)),
            pl.BlockSpec((1, 1, L), lambda j: (j, 0, 0)),
        ],
        out_shape=[
            jax.ShapeDtypeStruct((FG, 1, L), jnp.float32),
            jax.ShapeDtypeStruct((FG, 1, L), jnp.float32),
        ],
    )(ps1, pm1, ps2, pm2, cnt2)


def _tc_head_body(mean_ref, mx_ref, w1a, w1b, b1, w2, b2, out_ref, acc):
    j = pl.program_id(0)

    @pl.when(j == 0)
    def _():
        acc[...] = jnp.zeros_like(acc)

    acc[...] += mean_ref[...][0] @ w1a[...] + mx_ref[...][0] @ w1b[...]

    @pl.when(j == FG - 1)
    def _():
        p = jax.nn.relu(acc[...] + b1[...])
        out_ref[...] = (p @ w2[...] + b2[...])[:, 0]


def _tc_head(mean3, mx3, w1a, w1b, b1, w2, b2):
    return pl.pallas_call(
        _tc_head_body,
        grid=(FG,),
        in_specs=[
            pl.BlockSpec((1, C, 16), lambda j: (j, 0, 0)),
            pl.BlockSpec((1, C, 16), lambda j: (j, 0, 0)),
            pl.BlockSpec((16, F), lambda j: (j, 0)),
            pl.BlockSpec((16, F), lambda j: (j, 0)),
            pl.BlockSpec((F,), lambda j: (0,)),
            pl.BlockSpec((F, 1), lambda j: (0, 0)),
            pl.BlockSpec((1,), lambda j: (0,)),
        ],
        out_specs=pl.BlockSpec((C,), lambda j: (0,)),
        out_shape=jax.ShapeDtypeStruct((C,), jnp.float32),
        scratch_shapes=[pltpu.VMEM((C, F), jnp.float32)],
    )(mean3, mx3, w1a, w1b, b1, w2, b2)


def kernel(x, edge_index, community, multi_community_nodes, multi_community_index,
           emb1_W, emb1_b, emb2_W, emb2_b, emb3_W, emb3_b,
           conv1_W, conv1_b, conv2_W, conv2_b,
           lin1_W, lin1_b, lin2_W, lin2_b):
    src = edge_index[0]
    dst = edge_index[1]
    ones_rows = jnp.ones((KE, F), jnp.float32)
    zrows = jnp.zeros((ZR, F), jnp.float32)

    deg_rows, cnt_rows = _sc_counts(dst, community, ones_rows, zrows)
    g1, dinv = _tc_prep(x, emb1_W, emb1_b, emb2_W, emb2_b, emb3_W, emb3_b,
                        conv1_W, deg_rows)
    ap1 = _sc_agg(g1, src, dst, zrows).reshape(N, F)
    h1t, g2 = _tc_mid(ap1, g1, dinv, conv1_b, conv2_W)
    ap2 = _sc_agg(g2, src, dst, zrows).reshape(N, F)
    ps1, pm1 = _sc_pool(h1t.reshape(FG * N, 16), community)
    h2t = _tc_post(ap2, g2, dinv, conv2_b)
    ps2, pm2 = _sc_pool(h2t.reshape(FG * N, 16), community)
    L = C * 16
    cnt16 = cnt_rows[:, :, :16].reshape(NC, L)
    mean2, mx2 = _tc_merge(ps1.reshape(Q, FG, 1, L), pm1.reshape(Q, FG, 1, L),
                           ps2.reshape(Q, FG, 1, L), pm2.reshape(Q, FG, 1, L),
                           cnt16)
    out = _tc_head(mean2.reshape(FG, C, 16), mx2.reshape(FG, C, 16),
                   lin1_W[:F], lin1_W[F:], lin1_b, lin2_W, lin2_b)
    return out


# speedup vs baseline: 9.6539x; 1.5547x over previous
import functools

import jax
import jax.numpy as jnp
from jax import lax
from jax.experimental import pallas as pl
from jax.experimental.pallas import tpu as pltpu
from jax.experimental.pallas import tpu_sc as plsc

N = 10000
E = 320000
C = 1024
F = 128
NC = 2
NS = 16
NW = NC * NS
EPW = E // NW
KE = 80
NCH = EPW // KE
ZR = 1000
ZW = N // ZR
CPW = C // NW
W16 = 16

_mesh = plsc.VectorSubcoreMesh(core_axis_name="c", subcore_axis_name="s")


def _wid():
    return lax.axis_index("s") * NC + lax.axis_index("c")


def _sc_counts_body(dst_hbm, comm_hbm, ones_hbm, zer_hbm,
                    deg_out, cnt_out,
                    dvec0, dvec1, cvec, ones_v, accd, accc, semi0, semi1):
    c = lax.axis_index("c")
    s = lax.axis_index("s")
    wid = _wid()
    @pl.when(s < ZW)
    def _():
        pltpu.sync_copy(zer_hbm, accd.at[pl.ds(s * ZR, ZR)])
    pltpu.sync_copy(zer_hbm.at[pl.ds(0, C // NS)],
                    accc.at[pl.ds(s * (C // NS), C // NS)])
    pltpu.sync_copy(ones_hbm, ones_v)
    plsc.subcore_barrier()

    ebase = wid * EPW

    def pref(k, dv, sem):
        pltpu.async_copy(dst_hbm.at[pl.ds(ebase + k * KE, KE)], dv, sem)

    def wait_idx(dv, sem):
        pltpu.make_async_copy(dst_hbm.at[pl.ds(0, KE)], dv, sem).wait()

    pref(0, dvec0, semi0)

    def deg_step(k2, carry):
        k = 2 * k2
        wait_idx(dvec0, semi0)
        pref(k + 1, dvec1, semi1)
        pltpu.sync_copy(ones_v, accd.at[dvec0], add=True)
        wait_idx(dvec1, semi1)

        @pl.when(k + 2 < NCH)
        def _():
            pref(k + 2, dvec0, semi0)
        pltpu.sync_copy(ones_v, accd.at[dvec1], add=True)
        return carry
    lax.fori_loop(0, NCH // 2, deg_step, 0)

    wait_idx(dvec0, semi0)
    pltpu.sync_copy(ones_v, accd.at[dvec0], add=True)

    def cnt_step(k, carry):
        j = wid + k * NW

        @pl.when(j < NCH)
        def _():
            base = j * KE
            pltpu.sync_copy(comm_hbm.at[pl.ds(base, KE)], cvec)
            pltpu.sync_copy(ones_v, accc.at[cvec], add=True)
        return carry
    lax.fori_loop(0, (NCH + NW - 1) // NW, cnt_step, 0)

    plsc.subcore_barrier()

    @pl.when(s < ZW)
    def _():
        pltpu.sync_copy(accd.at[pl.ds(s * ZR, ZR)], deg_out.at[c, pl.ds(s * ZR, ZR)])
    pltpu.sync_copy(accc.at[pl.ds(s * (C // NS), C // NS)],
                    cnt_out.at[c, pl.ds(s * (C // NS), C // NS)])


_sc_counts = pl.kernel(
    _sc_counts_body,
    out_type=(jax.ShapeDtypeStruct((NC, N, F), jnp.float32),
              jax.ShapeDtypeStruct((NC, C, F), jnp.float32)),
    mesh=_mesh,
    scratch_types=[
        pltpu.VMEM((KE,), jnp.int32),
        pltpu.VMEM((KE,), jnp.int32),
        pltpu.VMEM((KE,), jnp.int32),
        pltpu.VMEM((KE, F), jnp.float32),
        pltpu.VMEM_SHARED((N, F), jnp.float32),
        pltpu.VMEM_SHARED((C, F), jnp.float32),
        pltpu.SemaphoreType.DMA,
        pltpu.SemaphoreType.DMA,
    ],
)


NH = N // NC
ACCR = NH + 8
EPS = E // NS
NCHA = EPS // KE


def _sc_agg_body(g_hbm, src_hbm, dst_hbm, zrows_hbm,
                 out_hbm,
                 src0, dst0, rows0, src1, dst1, rows1, acc,
                 semi0, semi1, semg):
    c = lax.axis_index("c")
    s = lax.axis_index("s")
    half = c * NH

    @pl.when(s < ZW // NC)
    def _():
        pltpu.sync_copy(zrows_hbm, acc.at[pl.ds(s * ZR, ZR)])

    @pl.when(s == ZW // NC)
    def _():
        pltpu.sync_copy(zrows_hbm.at[pl.ds(0, 8)], acc.at[pl.ds(NH, 8)])
    plsc.subcore_barrier()

    ebase = s * EPS

    def pref(k, sv, dv, sem):
        b = ebase + k * KE
        pltpu.async_copy(src_hbm.at[pl.ds(b, KE)], sv, sem)
        pltpu.async_copy(dst_hbm.at[pl.ds(b, KE)], dv, sem)

    def wait_idx(sv, dv, sem):
        pltpu.make_async_copy(src_hbm.at[pl.ds(0, KE)], sv, sem).wait()
        pltpu.make_async_copy(dst_hbm.at[pl.ds(0, KE)], dv, sem).wait()

    def remap(dv):
        for j in range(KE // 16):
            d16 = dv[pl.ds(j * 16, 16)] - half
            ok = (d16 >= 0) & (d16 < NH)
            dv[pl.ds(j * 16, 16)] = jnp.where(ok, d16, NH)

    pref(0, src0, dst0, semi0)

    def step(k2, carry):
        k = 2 * k2
        wait_idx(src0, dst0, semi0)
        remap(dst0)
        gd0 = pltpu.async_copy(g_hbm.at[src0], rows0, semg)
        pref(k + 1, src1, dst1, semi1)
        gd0.wait()
        wait_idx(src1, dst1, semi1)
        remap(dst1)
        gd1 = pltpu.async_copy(g_hbm.at[src1], rows1, semg)
        pltpu.sync_copy(rows0, acc.at[dst0], add=True)
        gd1.wait()

        @pl.when(k + 2 < NCHA)
        def _():
            pref(k + 2, src0, dst0, semi0)
        pltpu.sync_copy(rows1, acc.at[dst1], add=True)
        return carry
    lax.fori_loop(0, NCHA // 2, step, 0)

    plsc.subcore_barrier()

    @pl.when(s < ZW // NC)
    def _():
        pltpu.sync_copy(acc.at[pl.ds(s * ZR, ZR)], out_hbm.at[c, pl.ds(s * ZR, ZR)])


_sc_agg = pl.kernel(
    _sc_agg_body,
    out_type=jax.ShapeDtypeStruct((NC, NH, F), jnp.float32),
    mesh=_mesh,
    scratch_types=[
        pltpu.VMEM((KE,), jnp.int32),
        pltpu.VMEM((KE,), jnp.int32),
        pltpu.VMEM((KE, F), jnp.float32),
        pltpu.VMEM((KE,), jnp.int32),
        pltpu.VMEM((KE,), jnp.int32),
        pltpu.VMEM((KE, F), jnp.float32),
        pltpu.VMEM_SHARED((ACCR, F), jnp.float32),
        pltpu.SemaphoreType.DMA,
        pltpu.SemaphoreType.DMA,
        pltpu.SemaphoreType.DMA,
    ],
)


FG = F // 16
Q = NW // FG
CHK = 400
NCHKT = N // CHK
KPQ = (NCHKT + Q - 1) // Q


def _sc_pool_body(ht_hbm, comm_hbm,
                  psum_out, pmax_out,
                  block_v, cvec, asum, amax, sem):
    wid = _wid()
    fg = wid % FG
    q = wid // FG

    zer = jnp.zeros((16,), jnp.float32)
    ninf = jnp.full((16,), -jnp.inf, jnp.float32)

    def init_acc(i, carry):
        asum[pl.ds(i * 16, 16)] = zer
        amax[pl.ds(i * 16, 16)] = ninf
        return carry
    lax.fori_loop(0, C, init_acc, 0)

    for k in range(KPQ):
        j = q + k * Q

        @pl.when(j < NCHKT)
        def _():
            base = j * CHK
            pltpu.sync_copy(ht_hbm.at[pl.ds(fg * N + base, CHK)], block_v)
            pltpu.sync_copy(comm_hbm.at[pl.ds(base, CHK)], cvec)

            def group(g, carry):
                cv16 = cvec[pl.ds(g * 16, 16)]
                for l in range(16):
                    a = cv16[l] * 16
                    val = block_v[g * 16 + l]
                    asum[pl.ds(a, 16)] = asum[pl.ds(a, 16)] + val
                    amax[pl.ds(a, 16)] = jnp.maximum(amax[pl.ds(a, 16)], val)
                return carry
            lax.fori_loop(0, CHK // 16, group, 0)

    pltpu.sync_copy(asum, psum_out.at[q, fg])
    pltpu.sync_copy(amax, pmax_out.at[q, fg])


_sc_pool = pl.kernel(
    _sc_pool_body,
    out_type=(jax.ShapeDtypeStruct((Q, FG, C * 16), jnp.float32),
              jax.ShapeDtypeStruct((Q, FG, C * 16), jnp.float32)),
    mesh=_mesh,
    scratch_types=[
        pltpu.VMEM((CHK, 16), jnp.float32),
        pltpu.VMEM((CHK,), jnp.int32),
        pltpu.VMEM((C * 16,), jnp.float32),
        pltpu.VMEM((C * 16,), jnp.float32),
        pltpu.SemaphoreType.DMA,
    ],
)


RB = 1000
GRID = N // RB


def _tc_prep_body(x_ref, w1, b1, w2, b2, w3, b3, wc1, deg_ref, g1_ref, dinv_ref):
    xb = x_ref[...]
    x1 = jax.nn.relu(xb[:, :8] @ w1[...] + b1[...])
    x2 = jax.nn.relu(xb[:, 8:20] @ w2[...] + b2[...])
    h = jax.nn.relu(jnp.concatenate([x1, x2], axis=1) @ w3[...] + b3[...])
    hw = h @ wc1[...]
    d3 = deg_ref[...]
    deg = d3[0, :, 0] + d3[1, :, 0] + 1.0
    dv = lax.rsqrt(deg)
    dinv_ref[...] = dv[:, None]
    g1_ref[...] = hw * dv[:, None]


def _tc_prep(x, w1, b1, w2, b2, w3, b3, wc1, deg_rows):
    return pl.pallas_call(
        _tc_prep_body,
        grid=(GRID,),
        in_specs=[
            pl.BlockSpec((RB, 20), lambda i: (i, 0)),
            pl.BlockSpec((8, F), lambda i: (0, 0)),
            pl.BlockSpec((F,), lambda i: (0,)),
            pl.BlockSpec((12, F), lambda i: (0, 0)),
            pl.BlockSpec((F,), lambda i: (0,)),
            pl.BlockSpec((2 * F, 2 * F), lambda i: (0, 0)),
            pl.BlockSpec((2 * F,), lambda i: (0,)),
            pl.BlockSpec((2 * F, F), lambda i: (0, 0)),
            pl.BlockSpec((NC, RB, F), lambda i: (0, i, 0)),
        ],
        out_specs=[
            pl.BlockSpec((RB, F), lambda i: (i, 0)),
            pl.BlockSpec((RB, 1), lambda i: (i, 0)),
        ],
        out_shape=[
            jax.ShapeDtypeStruct((N, F), jnp.float32),
            jax.ShapeDtypeStruct((N, 1), jnp.float32),
        ],
    )(x, w1, b1, w2, b2, w3, b3, wc1, deg_rows)


def _tc_mid_body(ap_ref, g1_ref, dinv_ref, bc1, wc2, h1t_ref, g2_ref):
    dv = dinv_ref[...]
    h1 = jax.nn.relu(dv * (ap_ref[...] + g1_ref[...]) + bc1[...])
    for j in range(FG):
        h1t_ref[j] = h1[:, j * 16:(j + 1) * 16]
    g2_ref[...] = (h1 @ wc2[...]) * dv


def _tc_mid(ap, g1, dinv, bc1, wc2):
    return pl.pallas_call(
        _tc_mid_body,
        grid=(GRID,),
        in_specs=[
            pl.BlockSpec((RB, F), lambda i: (i, 0)),
            pl.BlockSpec((RB, F), lambda i: (i, 0)),
            pl.BlockSpec((RB, 1), lambda i: (i, 0)),
            pl.BlockSpec((F,), lambda i: (0,)),
            pl.BlockSpec((F, F), lambda i: (0, 0)),
        ],
        out_specs=[
            pl.BlockSpec((FG, RB, 16), lambda i: (0, i, 0)),
            pl.BlockSpec((RB, F), lambda i: (i, 0)),
        ],
        out_shape=[
            jax.ShapeDtypeStruct((FG, N, 16), jnp.float32),
            jax.ShapeDtypeStruct((N, F), jnp.float32),
        ],
    )(ap, g1, dinv, bc1, wc2)


def _tc_post_body(ap_ref, g2_ref, dinv_ref, bc2, h2t_ref):
    h2 = jax.nn.relu(dinv_ref[...] * (ap_ref[...] + g2_ref[...]) + bc2[...])
    for j in range(FG):
        h2t_ref[j] = h2[:, j * 16:(j + 1) * 16]


def _tc_post(ap2, g2, dinv, bc2):
    return pl.pallas_call(
        _tc_post_body,
        grid=(GRID,),
        in_specs=[
            pl.BlockSpec((RB, F), lambda i: (i, 0)),
            pl.BlockSpec((RB, F), lambda i: (i, 0)),
            pl.BlockSpec((RB, 1), lambda i: (i, 0)),
            pl.BlockSpec((F,), lambda i: (0,)),
        ],
        out_specs=pl.BlockSpec((FG, RB, 16), lambda i: (0, i, 0)),
        out_shape=jax.ShapeDtypeStruct((FG, N, 16), jnp.float32),
    )(ap2, g2, dinv, bc2)


def _tc_merge_body(ps1, pm1, ps2, pm2, cnt_ref, mean_ref, mx_ref):
    cr = cnt_ref[...]
    cntf = cr[0] + cr[1]
    s1 = ps1[...][:, 0, 0, :].sum(axis=0)
    s2 = ps2[...][:, 0, 0, :].sum(axis=0)
    m1 = pm1[...][:, 0, 0, :].max(axis=0)
    m2 = pm2[...][:, 0, 0, :].max(axis=0)
    nz = cntf > 0.0
    mean = (s1 + s2) / jnp.maximum(cntf, 1.0)
    mx = jnp.where(nz, m1, 0.0) + jnp.where(nz, m2, 0.0)
    mean_ref[...] = mean[None, None, :]
    mx_ref[...] = mx[None, None, :]


def _tc_merge(ps1, pm1, ps2, pm2, cnt2):
    L = C * 16
    return pl.pallas_call(
        _tc_merge_body,
        grid=(FG,),
        in_specs=[pl.BlockSpec((Q, 1, 1, L), lambda j: (0, j, 0, 0))] * 4 +
                 [pl.BlockSpec((NC, L), lambda j: (0, 0))],
        out_specs=[
            pl.BlockSpec((1, 1, L), lambda j: (j, 0, 0)),
            pl.BlockSpec((1, 1, L), lambda j: (j, 0, 0)),
        ],
        out_shape=[
            jax.ShapeDtypeStruct((FG, 1, L), jnp.float32),
            jax.ShapeDtypeStruct((FG, 1, L), jnp.float32),
        ],
    )(ps1, pm1, ps2, pm2, cnt2)


def _tc_head_body(mean_ref, mx_ref, w1a, w1b, b1, w2, b2, out_ref, acc):
    j = pl.program_id(0)

    @pl.when(j == 0)
    def _():
        acc[...] = jnp.zeros_like(acc)

    acc[...] += mean_ref[...][0] @ w1a[...] + mx_ref[...][0] @ w1b[...]

    @pl.when(j == FG - 1)
    def _():
        p = jax.nn.relu(acc[...] + b1[...])
        out_ref[...] = (p @ w2[...] + b2[...])[:, 0]


def _tc_head(mean3, mx3, w1a, w1b, b1, w2, b2):
    return pl.pallas_call(
        _tc_head_body,
        grid=(FG,),
        in_specs=[
            pl.BlockSpec((1, C, 16), lambda j: (j, 0, 0)),
            pl.BlockSpec((1, C, 16), lambda j: (j, 0, 0)),
            pl.BlockSpec((16, F), lambda j: (j, 0)),
            pl.BlockSpec((16, F), lambda j: (j, 0)),
            pl.BlockSpec((F,), lambda j: (0,)),
            pl.BlockSpec((F, 1), lambda j: (0, 0)),
            pl.BlockSpec((1,), lambda j: (0,)),
        ],
        out_specs=pl.BlockSpec((C,), lambda j: (0,)),
        out_shape=jax.ShapeDtypeStruct((C,), jnp.float32),
        scratch_shapes=[pltpu.VMEM((C, F), jnp.float32)],
    )(mean3, mx3, w1a, w1b, b1, w2, b2)


def kernel(x, edge_index, community, multi_community_nodes, multi_community_index,
           emb1_W, emb1_b, emb2_W, emb2_b, emb3_W, emb3_b,
           conv1_W, conv1_b, conv2_W, conv2_b,
           lin1_W, lin1_b, lin2_W, lin2_b):
    src = edge_index[0]
    dst = edge_index[1]
    ones_rows = jnp.ones((KE, F), jnp.float32)
    zrows = jnp.zeros((ZR, F), jnp.float32)

    deg_rows, cnt_rows = _sc_counts(dst, community, ones_rows, zrows)
    g1, dinv = _tc_prep(x, emb1_W, emb1_b, emb2_W, emb2_b, emb3_W, emb3_b,
                        conv1_W, deg_rows)
    ap1 = _sc_agg(g1, src, dst, zrows).reshape(N, F)
    h1t, g2 = _tc_mid(ap1, g1, dinv, conv1_b, conv2_W)
    ap2 = _sc_agg(g2, src, dst, zrows).reshape(N, F)
    ps1, pm1 = _sc_pool(h1t.reshape(FG * N, 16), community)
    h2t = _tc_post(ap2, g2, dinv, conv2_b)
    ps2, pm2 = _sc_pool(h2t.reshape(FG * N, 16), community)
    L = C * 16
    cnt16 = cnt_rows[:, :, :16].reshape(NC, L)
    mean2, mx2 = _tc_merge(ps1.reshape(Q, FG, 1, L), pm1.reshape(Q, FG, 1, L),
                           ps2.reshape(Q, FG, 1, L), pm2.reshape(Q, FG, 1, L),
                           cnt16)
    out = _tc_head(mean2.reshape(FG, C, 16), mx2.reshape(FG, C, 16),
                   lin1_W[:F], lin1_W[F:], lin1_b, lin2_W, lin2_b)
    return out


# speedup vs baseline: 11.7471x; 1.2168x over previous
import functools

import jax
import jax.numpy as jnp
from jax import lax
from jax.experimental import pallas as pl
from jax.experimental.pallas import tpu as pltpu
from jax.experimental.pallas import tpu_sc as plsc

N = 10000
E = 320000
C = 1024
F = 128
NC = 2
NS = 16
NW = NC * NS
EPW = E // NW
KE = 80
NCH = EPW // KE
ZR = 1000
ZW = N // ZR
CPW = C // NW
W16 = 16

_mesh = plsc.VectorSubcoreMesh(core_axis_name="c", subcore_axis_name="s")


def _wid():
    return lax.axis_index("s") * NC + lax.axis_index("c")


def _sc_counts_body(dst_hbm, comm_hbm, ones_hbm, zer_hbm,
                    deg_out, cnt_out,
                    dvec0, dvec1, cvec, ones_v, accd, accc, semi0, semi1):
    c = lax.axis_index("c")
    s = lax.axis_index("s")
    wid = _wid()
    @pl.when(s < ZW)
    def _():
        pltpu.sync_copy(zer_hbm, accd.at[pl.ds(s * ZR, ZR)])
    pltpu.sync_copy(zer_hbm.at[pl.ds(0, C // NS)],
                    accc.at[pl.ds(s * (C // NS), C // NS)])
    pltpu.sync_copy(ones_hbm, ones_v)
    plsc.subcore_barrier()

    ebase = wid * EPW

    def pref(k, dv, sem):
        pltpu.async_copy(dst_hbm.at[pl.ds(ebase + k * KE, KE)], dv, sem)

    def wait_idx(dv, sem):
        pltpu.make_async_copy(dst_hbm.at[pl.ds(0, KE)], dv, sem).wait()

    pref(0, dvec0, semi0)

    def deg_step(k2, carry):
        k = 2 * k2
        wait_idx(dvec0, semi0)
        pref(k + 1, dvec1, semi1)
        pltpu.sync_copy(ones_v, accd.at[dvec0], add=True)
        wait_idx(dvec1, semi1)

        @pl.when(k + 2 < NCH)
        def _():
            pref(k + 2, dvec0, semi0)
        pltpu.sync_copy(ones_v, accd.at[dvec1], add=True)
        return carry
    lax.fori_loop(0, NCH // 2, deg_step, 0)

    wait_idx(dvec0, semi0)
    pltpu.sync_copy(ones_v, accd.at[dvec0], add=True)

    def cnt_step(k, carry):
        j = wid + k * NW

        @pl.when(j < NCH)
        def _():
            base = j * KE
            pltpu.sync_copy(comm_hbm.at[pl.ds(base, KE)], cvec)
            pltpu.sync_copy(ones_v, accc.at[cvec], add=True)
        return carry
    lax.fori_loop(0, (NCH + NW - 1) // NW, cnt_step, 0)

    plsc.subcore_barrier()

    @pl.when(s < ZW)
    def _():
        pltpu.sync_copy(accd.at[pl.ds(s * ZR, ZR)], deg_out.at[c, pl.ds(s * ZR, ZR)])
    pltpu.sync_copy(accc.at[pl.ds(s * (C // NS), C // NS)],
                    cnt_out.at[c, pl.ds(s * (C // NS), C // NS)])


_sc_counts = pl.kernel(
    _sc_counts_body,
    out_type=(jax.ShapeDtypeStruct((NC, N, F), jnp.float32),
              jax.ShapeDtypeStruct((NC, C, F), jnp.float32)),
    mesh=_mesh,
    scratch_types=[
        pltpu.VMEM((KE,), jnp.int32),
        pltpu.VMEM((KE,), jnp.int32),
        pltpu.VMEM((KE,), jnp.int32),
        pltpu.VMEM((KE, F), jnp.float32),
        pltpu.VMEM_SHARED((N, F), jnp.float32),
        pltpu.VMEM_SHARED((C, F), jnp.float32),
        pltpu.SemaphoreType.DMA,
        pltpu.SemaphoreType.DMA,
    ],
)


NH = N // NC
ACCR = NH + 8
EPS = E // NS
KA = 128
NCHA = EPS // KA
KT = EPS - NCHA * KA
NSLOT = 3


def _sc_agg_body(g_hbm, src_hbm, dst_hbm, zrows_hbm,
                 out_hbm,
                 srcs, dsts, dstm, rows, tsrc, tdst, trows, acc,
                 semis, semg, sems):
    c = lax.axis_index("c")
    s = lax.axis_index("s")
    half = c * NH

    @pl.when(s < ZW // NC)
    def _():
        pltpu.sync_copy(zrows_hbm, acc.at[pl.ds(s * ZR, ZR)])

    @pl.when(s == ZW // NC)
    def _():
        pltpu.sync_copy(zrows_hbm.at[pl.ds(0, 8)], acc.at[pl.ds(NH, 8)])
    plsc.subcore_barrier()

    ebase = s * EPS

    def pref(k, x):
        b = ebase + k * KA
        pltpu.async_copy(src_hbm.at[pl.ds(b, KA)], srcs[x], semis[x])
        pltpu.async_copy(dst_hbm.at[pl.ds(b, KA)], dsts[x], semis[x])

    def wait_idx(x):
        pltpu.make_async_copy(src_hbm.at[pl.ds(0, KA)], srcs[x], semis[x]).wait()
        pltpu.make_async_copy(dst_hbm.at[pl.ds(0, KA)], dsts[x], semis[x]).wait()

    def remap(dv, dm, n):
        for j in range(n // 16):
            d16 = dv[pl.ds(j * 16, 16)] - half
            ok = (d16 >= 0) & (d16 < NH)
            dm[pl.ds(j * 16, 16)] = jnp.where(ok, d16, NH)

    def wait_scatter(x):
        pltpu.make_async_copy(g_hbm.at[pl.ds(0, KA)], rows[x], sems[x]).wait()

    for x in range(NSLOT):
        pref(x, x)

    def step(k3, carry):
        k = NSLOT * k3
        gds = []
        for x in range(NSLOT):
            @pl.when(k3 > 0)
            def _(x=x):
                wait_scatter(x)
            wait_idx(x)
            remap(dsts[x], dstm[x], KA)
            gds.append(pltpu.async_copy(g_hbm.at[srcs[x]], rows[x], semg))
        for x in range(NSLOT):
            gds[x].wait()

            @pl.when(k + NSLOT + x < NCHA)
            def _(k=k, x=x):
                pref(k + NSLOT + x, x)
            pltpu.async_copy(rows[x], acc.at[dstm[x]], sems[x], add=True)
        return carry
    lax.fori_loop(0, NCHA // NSLOT, step, 0)

    for x in range(NSLOT):
        wait_scatter(x)

    tb = ebase + NCHA * KA
    pltpu.sync_copy(src_hbm.at[pl.ds(tb, KT)], tsrc)
    pltpu.sync_copy(dst_hbm.at[pl.ds(tb, KT)], tdst)
    remap(tdst, tdst, KT)
    pltpu.async_copy(g_hbm.at[tsrc], trows, semg).wait()
    pltpu.sync_copy(trows, acc.at[tdst], add=True)

    plsc.subcore_barrier()

    @pl.when(s < ZW // NC)
    def _():
        pltpu.sync_copy(acc.at[pl.ds(s * ZR, ZR)], out_hbm.at[c, pl.ds(s * ZR, ZR)])


_sc_agg = pl.kernel(
    _sc_agg_body,
    out_type=jax.ShapeDtypeStruct((NC, NH, F), jnp.float32),
    mesh=_mesh,
    scratch_types=[
        [pltpu.VMEM((KA,), jnp.int32)] * NSLOT,
        [pltpu.VMEM((KA,), jnp.int32)] * NSLOT,
        [pltpu.VMEM((KA,), jnp.int32)] * NSLOT,
        [pltpu.VMEM((KA, F), jnp.float32)] * NSLOT,
        pltpu.VMEM((KT,), jnp.int32),
        pltpu.VMEM((KT,), jnp.int32),
        pltpu.VMEM((KT, F), jnp.float32),
        pltpu.VMEM_SHARED((ACCR, F), jnp.float32),
        [pltpu.SemaphoreType.DMA] * NSLOT,
        pltpu.SemaphoreType.DMA,
        [pltpu.SemaphoreType.DMA] * NSLOT,
    ],
)


FG = F // 16
Q = NW // FG
CHK = 400
NCHKT = N // CHK
KPQ = (NCHKT + Q - 1) // Q


def _sc_pool_body(ht_hbm, comm_hbm,
                  psum_out, pmax_out,
                  block_v, cvec, asum, amax, sem):
    wid = _wid()
    fg = wid % FG
    q = wid // FG

    zer = jnp.zeros((16,), jnp.float32)
    ninf = jnp.full((16,), -jnp.inf, jnp.float32)

    def init_acc(i, carry):
        asum[pl.ds(i * 16, 16)] = zer
        amax[pl.ds(i * 16, 16)] = ninf
        return carry
    lax.fori_loop(0, C, init_acc, 0)

    for k in range(KPQ):
        j = q + k * Q

        @pl.when(j < NCHKT)
        def _():
            base = j * CHK
            pltpu.sync_copy(ht_hbm.at[pl.ds(fg * N + base, CHK)], block_v)
            pltpu.sync_copy(comm_hbm.at[pl.ds(base, CHK)], cvec)

            def group(g, carry):
                cv16 = cvec[pl.ds(g * 16, 16)]
                for l in range(16):
                    a = cv16[l] * 16
                    val = block_v[g * 16 + l]
                    asum[pl.ds(a, 16)] = asum[pl.ds(a, 16)] + val
                    amax[pl.ds(a, 16)] = jnp.maximum(amax[pl.ds(a, 16)], val)
                return carry
            lax.fori_loop(0, CHK // 16, group, 0)

    pltpu.sync_copy(asum, psum_out.at[q, fg])
    pltpu.sync_copy(amax, pmax_out.at[q, fg])


_sc_pool = pl.kernel(
    _sc_pool_body,
    out_type=(jax.ShapeDtypeStruct((Q, FG, C * 16), jnp.float32),
              jax.ShapeDtypeStruct((Q, FG, C * 16), jnp.float32)),
    mesh=_mesh,
    scratch_types=[
        pltpu.VMEM((CHK, 16), jnp.float32),
        pltpu.VMEM((CHK,), jnp.int32),
        pltpu.VMEM((C * 16,), jnp.float32),
        pltpu.VMEM((C * 16,), jnp.float32),
        pltpu.SemaphoreType.DMA,
    ],
)


RB = 1000
GRID = N // RB


def _tc_prep_body(x_ref, w1, b1, w2, b2, w3, b3, wc1, deg_ref, g1_ref, dinv_ref):
    xb = x_ref[...]
    x1 = jax.nn.relu(xb[:, :8] @ w1[...] + b1[...])
    x2 = jax.nn.relu(xb[:, 8:20] @ w2[...] + b2[...])
    h = jax.nn.relu(jnp.concatenate([x1, x2], axis=1) @ w3[...] + b3[...])
    hw = h @ wc1[...]
    d3 = deg_ref[...]
    deg = d3[0, :, 0] + d3[1, :, 0] + 1.0
    dv = lax.rsqrt(deg)
    dinv_ref[...] = dv[:, None]
    g1_ref[...] = hw * dv[:, None]


def _tc_prep(x, w1, b1, w2, b2, w3, b3, wc1, deg_rows):
    return pl.pallas_call(
        _tc_prep_body,
        grid=(GRID,),
        in_specs=[
            pl.BlockSpec((RB, 20), lambda i: (i, 0)),
            pl.BlockSpec((8, F), lambda i: (0, 0)),
            pl.BlockSpec((F,), lambda i: (0,)),
            pl.BlockSpec((12, F), lambda i: (0, 0)),
            pl.BlockSpec((F,), lambda i: (0,)),
            pl.BlockSpec((2 * F, 2 * F), lambda i: (0, 0)),
            pl.BlockSpec((2 * F,), lambda i: (0,)),
            pl.BlockSpec((2 * F, F), lambda i: (0, 0)),
            pl.BlockSpec((NC, RB, F), lambda i: (0, i, 0)),
        ],
        out_specs=[
            pl.BlockSpec((RB, F), lambda i: (i, 0)),
            pl.BlockSpec((RB, 1), lambda i: (i, 0)),
        ],
        out_shape=[
            jax.ShapeDtypeStruct((N, F), jnp.float32),
            jax.ShapeDtypeStruct((N, 1), jnp.float32),
        ],
    )(x, w1, b1, w2, b2, w3, b3, wc1, deg_rows)


def _tc_mid_body(ap_ref, g1_ref, dinv_ref, bc1, wc2, h1t_ref, g2_ref):
    dv = dinv_ref[...]
    h1 = jax.nn.relu(dv * (ap_ref[...] + g1_ref[...]) + bc1[...])
    for j in range(FG):
        h1t_ref[j] = h1[:, j * 16:(j + 1) * 16]
    g2_ref[...] = (h1 @ wc2[...]) * dv


def _tc_mid(ap, g1, dinv, bc1, wc2):
    return pl.pallas_call(
        _tc_mid_body,
        grid=(GRID,),
        in_specs=[
            pl.BlockSpec((RB, F), lambda i: (i, 0)),
            pl.BlockSpec((RB, F), lambda i: (i, 0)),
            pl.BlockSpec((RB, 1), lambda i: (i, 0)),
            pl.BlockSpec((F,), lambda i: (0,)),
            pl.BlockSpec((F, F), lambda i: (0, 0)),
        ],
        out_specs=[
            pl.BlockSpec((FG, RB, 16), lambda i: (0, i, 0)),
            pl.BlockSpec((RB, F), lambda i: (i, 0)),
        ],
        out_shape=[
            jax.ShapeDtypeStruct((FG, N, 16), jnp.float32),
            jax.ShapeDtypeStruct((N, F), jnp.float32),
        ],
    )(ap, g1, dinv, bc1, wc2)


def _tc_post_body(ap_ref, g2_ref, dinv_ref, bc2, h2t_ref):
    h2 = jax.nn.relu(dinv_ref[...] * (ap_ref[...] + g2_ref[...]) + bc2[...])
    for j in range(FG):
        h2t_ref[j] = h2[:, j * 16:(j + 1) * 16]


def _tc_post(ap2, g2, dinv, bc2):
    return pl.pallas_call(
        _tc_post_body,
        grid=(GRID,),
        in_specs=[
            pl.BlockSpec((RB, F), lambda i: (i, 0)),
            pl.BlockSpec((RB, F), lambda i: (i, 0)),
            pl.BlockSpec((RB, 1), lambda i: (i, 0)),
            pl.BlockSpec((F,), lambda i: (0,)),
        ],
        out_specs=pl.BlockSpec((FG, RB, 16), lambda i: (0, i, 0)),
        out_shape=jax.ShapeDtypeStruct((FG, N, 16), jnp.float32),
    )(ap2, g2, dinv, bc2)


def _tc_merge_body(ps1, pm1, ps2, pm2, cnt_ref, mean_ref, mx_ref):
    cr = cnt_ref[...]
    cntf = cr[0] + cr[1]
    s1 = ps1[...][:, 0, 0, :].sum(axis=0)
    s2 = ps2[...][:, 0, 0, :].sum(axis=0)
    m1 = pm1[...][:, 0, 0, :].max(axis=0)
    m2 = pm2[...][:, 0, 0, :].max(axis=0)
    nz = cntf > 0.0
    mean = (s1 + s2) / jnp.maximum(cntf, 1.0)
    mx = jnp.where(nz, m1, 0.0) + jnp.where(nz, m2, 0.0)
    mean_ref[...] = mean[None, None, :]
    mx_ref[...] = mx[None, None, :]


def _tc_merge(ps1, pm1, ps2, pm2, cnt2):
    L = C * 16
    return pl.pallas_call(
        _tc_merge_body,
        grid=(FG,),
        in_specs=[pl.BlockSpec((Q, 1, 1, L), lambda j: (0, j, 0, 0))] * 4 +
                 [pl.BlockSpec((NC, L), lambda j: (0, 0))],
        out_specs=[
            pl.BlockSpec((1, 1, L), lambda j: (j, 0, 0)),
            pl.BlockSpec((1, 1, L), lambda j: (j, 0, 0)),
        ],
        out_shape=[
            jax.ShapeDtypeStruct((FG, 1, L), jnp.float32),
            jax.ShapeDtypeStruct((FG, 1, L), jnp.float32),
        ],
    )(ps1, pm1, ps2, pm2, cnt2)


def _tc_head_body(mean_ref, mx_ref, w1a, w1b, b1, w2, b2, out_ref, acc):
    j = pl.program_id(0)

    @pl.when(j == 0)
    def _():
        acc[...] = jnp.zeros_like(acc)

    acc[...] += mean_ref[...][0] @ w1a[...] + mx_ref[...][0] @ w1b[...]

    @pl.when(j == FG - 1)
    def _():
        p = jax.nn.relu(acc[...] + b1[...])
        out_ref[...] = (p @ w2[...] + b2[...])[:, 0]


def _tc_head(mean3, mx3, w1a, w1b, b1, w2, b2):
    return pl.pallas_call(
        _tc_head_body,
        grid=(FG,),
        in_specs=[
            pl.BlockSpec((1, C, 16), lambda j: (j, 0, 0)),
            pl.BlockSpec((1, C, 16), lambda j: (j, 0, 0)),
            pl.BlockSpec((16, F), lambda j: (j, 0)),
            pl.BlockSpec((16, F), lambda j: (j, 0)),
            pl.BlockSpec((F,), lambda j: (0,)),
            pl.BlockSpec((F, 1), lambda j: (0, 0)),
            pl.BlockSpec((1,), lambda j: (0,)),
        ],
        out_specs=pl.BlockSpec((C,), lambda j: (0,)),
        out_shape=jax.ShapeDtypeStruct((C,), jnp.float32),
        scratch_shapes=[pltpu.VMEM((C, F), jnp.float32)],
    )(mean3, mx3, w1a, w1b, b1, w2, b2)


def kernel(x, edge_index, community, multi_community_nodes, multi_community_index,
           emb1_W, emb1_b, emb2_W, emb2_b, emb3_W, emb3_b,
           conv1_W, conv1_b, conv2_W, conv2_b,
           lin1_W, lin1_b, lin2_W, lin2_b):
    src = edge_index[0]
    dst = edge_index[1]
    ones_rows = jnp.ones((KE, F), jnp.float32)
    zrows = jnp.zeros((ZR, F), jnp.float32)

    deg_rows, cnt_rows = _sc_counts(dst, community, ones_rows, zrows)
    g1, dinv = _tc_prep(x, emb1_W, emb1_b, emb2_W, emb2_b, emb3_W, emb3_b,
                        conv1_W, deg_rows)
    ap1 = _sc_agg(g1, src, dst, zrows).reshape(N, F)
    h1t, g2 = _tc_mid(ap1, g1, dinv, conv1_b, conv2_W)
    ap2 = _sc_agg(g2, src, dst, zrows).reshape(N, F)
    ps1, pm1 = _sc_pool(h1t.reshape(FG * N, 16), community)
    h2t = _tc_post(ap2, g2, dinv, conv2_b)
    ps2, pm2 = _sc_pool(h2t.reshape(FG * N, 16), community)
    L = C * 16
    cnt16 = cnt_rows[:, :, :16].reshape(NC, L)
    mean2, mx2 = _tc_merge(ps1.reshape(Q, FG, 1, L), pm1.reshape(Q, FG, 1, L),
                           ps2.reshape(Q, FG, 1, L), pm2.reshape(Q, FG, 1, L),
                           cnt16)
    out = _tc_head(mean2.reshape(FG, C, 16), mx2.reshape(FG, C, 16),
                   lin1_W[:F], lin1_W[F:], lin1_b, lin2_W, lin2_b)
    return out


# speedup vs baseline: 11.7529x; 1.0005x over previous
import jax
import jax.numpy as jnp
from jax import lax
from jax.experimental import pallas as pl
from jax.experimental.pallas import tpu as pltpu
from jax.experimental.pallas import tpu_sc as plsc

N = 10000
E = 320000
C = 1024
F = 128
NC = 2
NS = 16
NW = NC * NS
EPW = E // NW
KE = 80
NCH = EPW // KE
ZR = 1000
ZW = N // ZR
CPW = C // NW
W16 = 16

_mesh = plsc.VectorSubcoreMesh(core_axis_name="c", subcore_axis_name="s")


def _wid():
    return lax.axis_index("s") * NC + lax.axis_index("c")


def _sc_counts_body(dst_hbm, comm_hbm, ones_hbm, zer_hbm,
                    deg_out, cnt_out,
                    dvec0, dvec1, cvec, ones_v, accd, accc, semi0, semi1):
    c = lax.axis_index("c")
    s = lax.axis_index("s")
    wid = _wid()
    @pl.when(s < ZW)
    def _():
        pltpu.sync_copy(zer_hbm, accd.at[pl.ds(s * ZR, ZR)])
    pltpu.sync_copy(zer_hbm.at[pl.ds(0, C // NS)],
                    accc.at[pl.ds(s * (C // NS), C // NS)])
    pltpu.sync_copy(ones_hbm, ones_v)
    plsc.subcore_barrier()

    ebase = wid * EPW

    def pref(k, dv, sem):
        pltpu.async_copy(dst_hbm.at[pl.ds(ebase + k * KE, KE)], dv, sem)

    def wait_idx(dv, sem):
        pltpu.make_async_copy(dst_hbm.at[pl.ds(0, KE)], dv, sem).wait()

    pref(0, dvec0, semi0)

    def deg_step(k2, carry):
        k = 2 * k2
        wait_idx(dvec0, semi0)
        pref(k + 1, dvec1, semi1)
        pltpu.sync_copy(ones_v, accd.at[dvec0], add=True)
        wait_idx(dvec1, semi1)

        @pl.when(k + 2 < NCH)
        def _():
            pref(k + 2, dvec0, semi0)
        pltpu.sync_copy(ones_v, accd.at[dvec1], add=True)
        return carry
    lax.fori_loop(0, NCH // 2, deg_step, 0)

    wait_idx(dvec0, semi0)
    pltpu.sync_copy(ones_v, accd.at[dvec0], add=True)

    def cnt_step(k, carry):
        j = wid + k * NW

        @pl.when(j < NCH)
        def _():
            base = j * KE
            pltpu.sync_copy(comm_hbm.at[pl.ds(base, KE)], cvec)
            pltpu.sync_copy(ones_v, accc.at[cvec], add=True)
        return carry
    lax.fori_loop(0, (NCH + NW - 1) // NW, cnt_step, 0)

    plsc.subcore_barrier()

    @pl.when(s < ZW)
    def _():
        pltpu.sync_copy(accd.at[pl.ds(s * ZR, ZR)], deg_out.at[c, pl.ds(s * ZR, ZR)])
    pltpu.sync_copy(accc.at[pl.ds(s * (C // NS), C // NS)],
                    cnt_out.at[c, pl.ds(s * (C // NS), C // NS)])


_sc_counts = pl.kernel(
    _sc_counts_body,
    out_type=(jax.ShapeDtypeStruct((NC, N, F), jnp.float32),
              jax.ShapeDtypeStruct((NC, C, F), jnp.float32)),
    mesh=_mesh,
    scratch_types=[
        pltpu.VMEM((KE,), jnp.int32),
        pltpu.VMEM((KE,), jnp.int32),
        pltpu.VMEM((KE,), jnp.int32),
        pltpu.VMEM((KE, F), jnp.float32),
        pltpu.VMEM_SHARED((N, F), jnp.float32),
        pltpu.VMEM_SHARED((C, F), jnp.float32),
        pltpu.SemaphoreType.DMA,
        pltpu.SemaphoreType.DMA,
    ],
)


NH = N // NC
ACCR = NH + 8
EPS = E // NS
KA = 128
NCHA = EPS // KA
KT = EPS - NCHA * KA
NSLOT = 3


def _sc_agg_body(g_hbm, src_hbm, dst_hbm, zrows_hbm,
                 out_hbm,
                 srcs, dsts, dstm, rows, tsrc, tdst, trows, acc,
                 semis, semg, sems):
    c = lax.axis_index("c")
    s = lax.axis_index("s")
    half = c * NH

    @pl.when(s < ZW // NC)
    def _():
        pltpu.sync_copy(zrows_hbm, acc.at[pl.ds(s * ZR, ZR)])

    @pl.when(s == ZW // NC)
    def _():
        pltpu.sync_copy(zrows_hbm.at[pl.ds(0, 8)], acc.at[pl.ds(NH, 8)])
    plsc.subcore_barrier()

    ebase = s * EPS

    def pref(k, x):
        b = ebase + k * KA
        pltpu.async_copy(src_hbm.at[pl.ds(b, KA)], srcs[x], semis[x])
        pltpu.async_copy(dst_hbm.at[pl.ds(b, KA)], dsts[x], semis[x])

    def wait_idx(x):
        pltpu.make_async_copy(src_hbm.at[pl.ds(0, KA)], srcs[x], semis[x]).wait()
        pltpu.make_async_copy(dst_hbm.at[pl.ds(0, KA)], dsts[x], semis[x]).wait()

    def remap(dv, dm, n):
        for j in range(n // 16):
            d16 = dv[pl.ds(j * 16, 16)] - half
            ok = (d16 >= 0) & (d16 < NH)
            dm[pl.ds(j * 16, 16)] = jnp.where(ok, d16, NH)

    def wait_scatter(x):
        pltpu.make_async_copy(g_hbm.at[pl.ds(0, KA)], rows[x], sems[x]).wait()

    for x in range(NSLOT):
        pref(x, x)

    def step(k3, carry):
        k = NSLOT * k3
        gds = []
        for x in range(NSLOT):
            @pl.when(k3 > 0)
            def _(x=x):
                wait_scatter(x)
            wait_idx(x)
            remap(dsts[x], dstm[x], KA)
            gds.append(pltpu.async_copy(g_hbm.at[srcs[x]], rows[x], semg))
        for x in range(NSLOT):
            gds[x].wait()

            @pl.when(k + NSLOT + x < NCHA)
            def _(k=k, x=x):
                pref(k + NSLOT + x, x)
            pltpu.async_copy(rows[x], acc.at[dstm[x]], sems[x], add=True)
        return carry
    lax.fori_loop(0, NCHA // NSLOT, step, 0)

    for x in range(NSLOT):
        wait_scatter(x)

    tb = ebase + NCHA * KA
    pltpu.sync_copy(src_hbm.at[pl.ds(tb, KT)], tsrc)
    pltpu.sync_copy(dst_hbm.at[pl.ds(tb, KT)], tdst)
    remap(tdst, tdst, KT)
    pltpu.async_copy(g_hbm.at[tsrc], trows, semg).wait()
    pltpu.sync_copy(trows, acc.at[tdst], add=True)

    plsc.subcore_barrier()

    @pl.when(s < ZW // NC)
    def _():
        pltpu.sync_copy(acc.at[pl.ds(s * ZR, ZR)], out_hbm.at[c, pl.ds(s * ZR, ZR)])


_sc_agg = pl.kernel(
    _sc_agg_body,
    out_type=jax.ShapeDtypeStruct((NC, NH, F), jnp.float32),
    mesh=_mesh,
    scratch_types=[
        [pltpu.VMEM((KA,), jnp.int32)] * NSLOT,
        [pltpu.VMEM((KA,), jnp.int32)] * NSLOT,
        [pltpu.VMEM((KA,), jnp.int32)] * NSLOT,
        [pltpu.VMEM((KA, F), jnp.float32)] * NSLOT,
        pltpu.VMEM((KT,), jnp.int32),
        pltpu.VMEM((KT,), jnp.int32),
        pltpu.VMEM((KT, F), jnp.float32),
        pltpu.VMEM_SHARED((ACCR, F), jnp.float32),
        [pltpu.SemaphoreType.DMA] * NSLOT,
        pltpu.SemaphoreType.DMA,
        [pltpu.SemaphoreType.DMA] * NSLOT,
    ],
)


FG = F // 16
Q = NW // FG
CHK = 400
NCHKT = N // CHK
KPQ = (NCHKT + Q - 1) // Q


def _sc_pool_body(ht_hbm, comm_hbm,
                  psum_out, pmax_out,
                  block_v, cvec, asum, amax, sem):
    wid = _wid()
    fg = wid % FG
    q = wid // FG

    zer = jnp.zeros((16,), jnp.float32)
    ninf = jnp.full((16,), -jnp.inf, jnp.float32)

    def init_acc(i, carry):
        asum[pl.ds(i * 16, 16)] = zer
        amax[pl.ds(i * 16, 16)] = ninf
        return carry
    lax.fori_loop(0, C, init_acc, 0)

    for k in range(KPQ):
        j = q + k * Q

        @pl.when(j < NCHKT)
        def _():
            base = j * CHK
            pltpu.sync_copy(ht_hbm.at[pl.ds(fg * N + base, CHK)], block_v)
            pltpu.sync_copy(comm_hbm.at[pl.ds(base, CHK)], cvec)

            def group(g, carry):
                cv16 = cvec[pl.ds(g * 16, 16)]
                for l in range(16):
                    a = cv16[l] * 16
                    val = block_v[g * 16 + l]
                    asum[pl.ds(a, 16)] = asum[pl.ds(a, 16)] + val
                    amax[pl.ds(a, 16)] = jnp.maximum(amax[pl.ds(a, 16)], val)
                return carry
            lax.fori_loop(0, CHK // 16, group, 0)

    pltpu.sync_copy(asum, psum_out.at[q, fg])
    pltpu.sync_copy(amax, pmax_out.at[q, fg])


_sc_pool = pl.kernel(
    _sc_pool_body,
    out_type=(jax.ShapeDtypeStruct((Q, FG, C * 16), jnp.float32),
              jax.ShapeDtypeStruct((Q, FG, C * 16), jnp.float32)),
    mesh=_mesh,
    scratch_types=[
        pltpu.VMEM((CHK, 16), jnp.float32),
        pltpu.VMEM((CHK,), jnp.int32),
        pltpu.VMEM((C * 16,), jnp.float32),
        pltpu.VMEM((C * 16,), jnp.float32),
        pltpu.SemaphoreType.DMA,
    ],
)


RB = 1000
GRID = N // RB


def _tc_prep_body(x_ref, w1, b1, w2, b2, w3, b3, wc1, deg_ref, g1_ref, dinv_ref):
    xb = x_ref[...]
    x1 = jax.nn.relu(xb[:, :8] @ w1[...] + b1[...])
    x2 = jax.nn.relu(xb[:, 8:20] @ w2[...] + b2[...])
    h = jax.nn.relu(jnp.concatenate([x1, x2], axis=1) @ w3[...] + b3[...])
    hw = h @ wc1[...]
    d3 = deg_ref[...]
    deg = d3[0, :, 0] + d3[1, :, 0] + 1.0
    dv = lax.rsqrt(deg)
    dinv_ref[...] = dv[:, None]
    g1_ref[...] = hw * dv[:, None]


def _tc_prep(x, w1, b1, w2, b2, w3, b3, wc1, deg_rows):
    return pl.pallas_call(
        _tc_prep_body,
        grid=(GRID,),
        in_specs=[
            pl.BlockSpec((RB, 20), lambda i: (i, 0)),
            pl.BlockSpec((8, F), lambda i: (0, 0)),
            pl.BlockSpec((F,), lambda i: (0,)),
            pl.BlockSpec((12, F), lambda i: (0, 0)),
            pl.BlockSpec((F,), lambda i: (0,)),
            pl.BlockSpec((2 * F, 2 * F), lambda i: (0, 0)),
            pl.BlockSpec((2 * F,), lambda i: (0,)),
            pl.BlockSpec((2 * F, F), lambda i: (0, 0)),
            pl.BlockSpec((NC, RB, F), lambda i: (0, i, 0)),
        ],
        out_specs=[
            pl.BlockSpec((RB, F), lambda i: (i, 0)),
            pl.BlockSpec((RB, 1), lambda i: (i, 0)),
        ],
        out_shape=[
            jax.ShapeDtypeStruct((N, F), jnp.float32),
            jax.ShapeDtypeStruct((N, 1), jnp.float32),
        ],
    )(x, w1, b1, w2, b2, w3, b3, wc1, deg_rows)


def _tc_mid_body(ap_ref, g1_ref, dinv_ref, bc1, wc2, h1t_ref, g2_ref):
    dv = dinv_ref[...]
    h1 = jax.nn.relu(dv * (ap_ref[...] + g1_ref[...]) + bc1[...])
    for j in range(FG):
        h1t_ref[j] = h1[:, j * 16:(j + 1) * 16]
    g2_ref[...] = (h1 @ wc2[...]) * dv


def _tc_mid(ap, g1, dinv, bc1, wc2):
    return pl.pallas_call(
        _tc_mid_body,
        grid=(GRID,),
        in_specs=[
            pl.BlockSpec((RB, F), lambda i: (i, 0)),
            pl.BlockSpec((RB, F), lambda i: (i, 0)),
            pl.BlockSpec((RB, 1), lambda i: (i, 0)),
            pl.BlockSpec((F,), lambda i: (0,)),
            pl.BlockSpec((F, F), lambda i: (0, 0)),
        ],
        out_specs=[
            pl.BlockSpec((FG, RB, 16), lambda i: (0, i, 0)),
            pl.BlockSpec((RB, F), lambda i: (i, 0)),
        ],
        out_shape=[
            jax.ShapeDtypeStruct((FG, N, 16), jnp.float32),
            jax.ShapeDtypeStruct((N, F), jnp.float32),
        ],
    )(ap, g1, dinv, bc1, wc2)


def _tc_post_body(ap_ref, g2_ref, dinv_ref, bc2, h2t_ref):
    h2 = jax.nn.relu(dinv_ref[...] * (ap_ref[...] + g2_ref[...]) + bc2[...])
    for j in range(FG):
        h2t_ref[j] = h2[:, j * 16:(j + 1) * 16]


def _tc_post(ap2, g2, dinv, bc2):
    return pl.pallas_call(
        _tc_post_body,
        grid=(GRID,),
        in_specs=[
            pl.BlockSpec((RB, F), lambda i: (i, 0)),
            pl.BlockSpec((RB, F), lambda i: (i, 0)),
            pl.BlockSpec((RB, 1), lambda i: (i, 0)),
            pl.BlockSpec((F,), lambda i: (0,)),
        ],
        out_specs=pl.BlockSpec((FG, RB, 16), lambda i: (0, i, 0)),
        out_shape=jax.ShapeDtypeStruct((FG, N, 16), jnp.float32),
    )(ap2, g2, dinv, bc2)


def _tc_merge_body(ps1, pm1, ps2, pm2, cnt_ref, mean_ref, mx_ref):
    cr = cnt_ref[...]
    cntf = cr[0] + cr[1]
    s1 = ps1[...][:, 0, 0, :].sum(axis=0)
    s2 = ps2[...][:, 0, 0, :].sum(axis=0)
    m1 = pm1[...][:, 0, 0, :].max(axis=0)
    m2 = pm2[...][:, 0, 0, :].max(axis=0)
    nz = cntf > 0.0
    mean = (s1 + s2) / jnp.maximum(cntf, 1.0)
    mx = jnp.where(nz, m1, 0.0) + jnp.where(nz, m2, 0.0)
    mean_ref[...] = mean[None, None, :]
    mx_ref[...] = mx[None, None, :]


def _tc_merge(ps1, pm1, ps2, pm2, cnt2):
    L = C * 16
    return pl.pallas_call(
        _tc_merge_body,
        grid=(FG,),
        in_specs=[pl.BlockSpec((Q, 1, 1, L), lambda j: (0, j, 0, 0))] * 4 +
                 [pl.BlockSpec((NC, L), lambda j: (0, 0))],
        out_specs=[
            pl.BlockSpec((1, 1, L), lambda j: (j, 0, 0)),
            pl.BlockSpec((1, 1, L), lambda j: (j, 0, 0)),
        ],
        out_shape=[
            jax.ShapeDtypeStruct((FG, 1, L), jnp.float32),
            jax.ShapeDtypeStruct((FG, 1, L), jnp.float32),
        ],
    )(ps1, pm1, ps2, pm2, cnt2)


def _tc_head_body(mean_ref, mx_ref, w1a, w1b, b1, w2, b2, out_ref, acc):
    j = pl.program_id(0)

    @pl.when(j == 0)
    def _():
        acc[...] = jnp.zeros_like(acc)

    acc[...] += mean_ref[...][0] @ w1a[...] + mx_ref[...][0] @ w1b[...]

    @pl.when(j == FG - 1)
    def _():
        p = jax.nn.relu(acc[...] + b1[...])
        out_ref[...] = (p @ w2[...] + b2[...])[:, 0]


def _tc_head(mean3, mx3, w1a, w1b, b1, w2, b2):
    return pl.pallas_call(
        _tc_head_body,
        grid=(FG,),
        in_specs=[
            pl.BlockSpec((1, C, 16), lambda j: (j, 0, 0)),
            pl.BlockSpec((1, C, 16), lambda j: (j, 0, 0)),
            pl.BlockSpec((16, F), lambda j: (j, 0)),
            pl.BlockSpec((16, F), lambda j: (j, 0)),
            pl.BlockSpec((F,), lambda j: (0,)),
            pl.BlockSpec((F, 1), lambda j: (0, 0)),
            pl.BlockSpec((1,), lambda j: (0,)),
        ],
        out_specs=pl.BlockSpec((C,), lambda j: (0,)),
        out_shape=jax.ShapeDtypeStruct((C,), jnp.float32),
        scratch_shapes=[pltpu.VMEM((C, F), jnp.float32)],
    )(mean3, mx3, w1a, w1b, b1, w2, b2)


def kernel(x, edge_index, community, multi_community_nodes, multi_community_index,
           emb1_W, emb1_b, emb2_W, emb2_b, emb3_W, emb3_b,
           conv1_W, conv1_b, conv2_W, conv2_b,
           lin1_W, lin1_b, lin2_W, lin2_b):
    src = edge_index[0]
    dst = edge_index[1]
    ones_rows = jnp.ones((KE, F), jnp.float32)
    zrows = jnp.zeros((ZR, F), jnp.float32)

    deg_rows, cnt_rows = _sc_counts(dst, community, ones_rows, zrows)
    g1, dinv = _tc_prep(x, emb1_W, emb1_b, emb2_W, emb2_b, emb3_W, emb3_b,
                        conv1_W, deg_rows)
    ap1 = _sc_agg(g1, src, dst, zrows).reshape(N, F)
    h1t, g2 = _tc_mid(ap1, g1, dinv, conv1_b, conv2_W)
    ap2 = _sc_agg(g2, src, dst, zrows).reshape(N, F)
    ps1, pm1 = _sc_pool(h1t.reshape(FG * N, 16), community)
    h2t = _tc_post(ap2, g2, dinv, conv2_b)
    ps2, pm2 = _sc_pool(h2t.reshape(FG * N, 16), community)
    L = C * 16
    cnt16 = cnt_rows[:, :, :16].reshape(NC, L)
    mean2, mx2 = _tc_merge(ps1.reshape(Q, FG, 1, L), pm1.reshape(Q, FG, 1, L),
                           ps2.reshape(Q, FG, 1, L), pm2.reshape(Q, FG, 1, L),
                           cnt16)
    out = _tc_head(mean2.reshape(FG, C, 16), mx2.reshape(FG, C, 16),
                   lin1_W[:F], lin1_W[F:], lin1_b, lin2_W, lin2_b)
    return out
